# baseline (speedup 1.0000x reference)
import jax
import jax.numpy as jnp
from jax import lax
from jax.experimental import pallas as pl
from jax.experimental.pallas import tpu as pltpu
from jax.experimental.pallas import tpu_sc as plsc

V = 1000
B, T = 1024, 50
N = B * T
NC, NS, L = 2, 16, 16
NW = NC * NS
BPW = B // NW
CH = T
CHP = 56
NG = BPW // 2
IDXW = BPW * CHP


def _lse_body(table_ref, out_ref):
    t = table_ref[...]
    m = jnp.max(t, axis=1)
    s = jnp.sum(jnp.exp(t - m[:, None]), axis=1)
    out_ref[...] = m + jnp.log(s)


def _row_lse(table):
    return pl.pallas_call(
        _lse_body,
        out_shape=jax.ShapeDtypeStruct((V,), jnp.float32),
    )(table)


def _loss_chunk(rows_v, idx_v, tgt_v, lse_v, off, acc):
    for i in range(0, CH - L + 1, L):
        row_ids = lax.iota(jnp.int32, L) + i
        idx16 = idx_v[pl.ds(off + i, L)]
        tgt16 = tgt_v[pl.ds(off + i, L)]
        lse16 = plsc.load_gather(lse_v, [idx16])
        x16 = plsc.load_gather(rows_v, [row_ids, tgt16])
        acc = acc + (lse16 - x16)
    rem = CH % L
    if rem:
        i = CH - rem
        lanes = lax.iota(jnp.int32, L)
        msk = lanes < rem
        row_ids = jnp.minimum(lanes + i, CH - 1)
        idx16 = idx_v[pl.ds(off + i, L)]
        tgt16 = tgt_v[pl.ds(off + i, L)]
        lse16 = plsc.load_gather(lse_v, [jnp.where(msk, idx16, 0)])
        x16 = plsc.load_gather(rows_v, [row_ids, jnp.where(msk, tgt16, 0)])
        acc = acc + jnp.where(msk, lse16 - x16, 0.0)
    return acc


def _sc_body(idx_hbm, tgt_hbm, table_hbm, lse_hbm, out_hbm, part_hbm,
             idx_v, tgt_v, lse_v, rows_a, rows_b, acc_v,
             gsem_a, gsem_b, ssem_a, ssem_b):
    wid = lax.axis_index("s") * NC + lax.axis_index("c")
    base = wid * IDXW
    bb = wid * BPW
    pltpu.sync_copy(idx_hbm.at[pl.ds(base, IDXW)], idx_v.at[pl.ds(0, IDXW)])
    pltpu.sync_copy(tgt_hbm.at[pl.ds(base, IDXW)], tgt_v.at[pl.ds(0, IDXW)])
    pltpu.sync_copy(lse_hbm, lse_v)

    def gather(c, buf, sem):
        pltpu.async_copy(table_hbm.at[idx_v.at[pl.ds(c * CHP, CH)]], buf, sem)

    def scatter(c, buf, sem):
        pltpu.async_copy(buf, out_hbm.at[bb + c], sem)

    def gather_wait(buf, sem):
        pltpu.make_async_copy(table_hbm.at[idx_v.at[pl.ds(0, CH)]], buf,
                              sem).wait()

    def scatter_wait(buf, sem):
        pltpu.make_async_copy(buf, out_hbm.at[bb], sem).wait()

    gather(0, rows_a, gsem_a)
    gather(1, rows_b, gsem_b)

    def group(g, acc):
        a = 2 * g
        gather_wait(rows_a, gsem_a)
        scatter(a, rows_a, ssem_a)
        acc = _loss_chunk(rows_a, idx_v, tgt_v, lse_v, a * CHP, acc)
        gather_wait(rows_b, gsem_b)
        scatter(a + 1, rows_b, ssem_b)
        acc = _loss_chunk(rows_b, idx_v, tgt_v, lse_v, (a + 1) * CHP, acc)
        scatter_wait(rows_a, ssem_a)
        gather(a + 2, rows_a, gsem_a)
        scatter_wait(rows_b, ssem_b)
        gather(a + 3, rows_b, gsem_b)
        return acc

    acc = lax.fori_loop(0, NG - 1, group, jnp.zeros((L,), jnp.float32))

    a = 2 * (NG - 1)
    gather_wait(rows_a, gsem_a)
    scatter(a, rows_a, ssem_a)
    acc = _loss_chunk(rows_a, idx_v, tgt_v, lse_v, a * CHP, acc)
    gather_wait(rows_b, gsem_b)
    scatter(a + 1, rows_b, ssem_b)
    acc = _loss_chunk(rows_b, idx_v, tgt_v, lse_v, (a + 1) * CHP, acc)
    scatter_wait(rows_a, ssem_a)
    scatter_wait(rows_b, ssem_b)

    acc_v[...] = acc
    pltpu.sync_copy(acc_v, part_hbm.at[wid])


def _sc_gather(idx_p, tgt_p, table, lse):
    mesh = plsc.VectorSubcoreMesh(core_axis_name="c", subcore_axis_name="s",
                                  num_cores=NC, num_subcores=NS)
    f = pl.kernel(
        _sc_body,
        out_type=(jax.ShapeDtypeStruct((B, T, V), jnp.float32),
                  jax.ShapeDtypeStruct((NW, L), jnp.float32)),
        mesh=mesh,
        scratch_types=[
            pltpu.VMEM((IDXW + L,), jnp.int32),
            pltpu.VMEM((IDXW + L,), jnp.int32),
            pltpu.VMEM((V,), jnp.float32),
            pltpu.VMEM((CH, V), jnp.float32),
            pltpu.VMEM((CH, V), jnp.float32),
            pltpu.VMEM((L,), jnp.float32),
            pltpu.SemaphoreType.DMA,
            pltpu.SemaphoreType.DMA,
            pltpu.SemaphoreType.DMA,
            pltpu.SemaphoreType.DMA,
        ],
        compiler_params=pltpu.CompilerParams(needs_layout_passes=False,
                                             use_tc_tiling_on_sc=False),
    )
    return f(idx_p, tgt_p, table, lse)


def kernel(idx, targets, table):
    pad = ((0, 0), (0, CHP - T))
    idx_p = jnp.pad(idx.astype(jnp.int32), pad).reshape(-1)
    tgt_p = jnp.pad(targets.astype(jnp.int32), pad).reshape(-1)
    lse = _row_lse(table)
    logits, parts = _sc_gather(idx_p, tgt_p, table, lse)
    loss = parts.sum() / jnp.float32(N)
    return (logits, loss)
